# Initial kernel scaffold; baseline (speedup 1.0000x reference)
import jax
import jax.numpy as jnp
from jax.experimental import pallas as pl


def kernel(x, edge_index, W1, b1, W2, b2):
    raise NotImplementedError("write your pallas kernel here")


# speedup vs baseline: 142.5030x; 142.5030x over previous
import functools

import jax
import jax.numpy as jnp
from jax import lax
from jax.experimental import pallas as pl
from jax.experimental.pallas import tpu as pltpu
from jax.experimental.pallas import tpu_sc as plsc

N_NODES = 100000
N_EDGES = 6400000

NC = 2
NS = 16
NW = NC * NS
CH = 128
EPT = 200704
E_PAD = EPT * NW
N_PAD = 100352
SLICE = N_PAD // NS
PAD_NODE = N_NODES
E_ROWS = E_PAD // CH

KI1 = 16
NBLK1 = EPT // (KI1 * CH)
KI3 = 8
NBLK3 = EPT // (KI3 * CH)

_f32 = jnp.float32


def _sc_deg_body(dst_hbm, ones_hbm, zeros_hbm, degp_hbm,
                 acc_sh, idx_v, ones_v, stage_v, ssem):
    c = lax.axis_index("c")
    s = lax.axis_index("s")
    wid = c * NS + s
    sl = pl.ds(s * SLICE, SLICE)
    pltpu.sync_copy(ones_hbm, ones_v)
    pltpu.sync_copy(zeros_hbm.at[sl], stage_v)
    pltpu.sync_copy(stage_v, acc_sh.at[sl])
    plsc.subcore_barrier()

    base_row = wid * (EPT // CH)

    def blk(g, _):
        pltpu.sync_copy(dst_hbm.at[pl.ds(base_row + g * KI1, KI1)], idx_v)
        cps = [
            pltpu.async_copy(ones_v, acc_sh.at[idx_v.at[j]], ssem, add=True)
            for j in range(KI1)
        ]
        for cp in cps:
            cp.wait()
        return ()

    lax.fori_loop(0, NBLK1, blk, (), unroll=False)
    plsc.subcore_barrier()
    pltpu.sync_copy(acc_sh.at[sl], stage_v)
    pltpu.sync_copy(stage_v, degp_hbm.at[c, sl])


def _sc_edge1_body(src_hbm, dst_hbm, tab_hbm, zeros_hbm, accp_hbm,
                   tab_sh, acc_sh, sidx_v, didx_v, val_v, stage_v, gsem, ssem):
    c = lax.axis_index("c")
    s = lax.axis_index("s")
    wid = c * NS + s
    sl = pl.ds(s * SLICE, SLICE)
    pltpu.sync_copy(tab_hbm.at[sl], stage_v)
    pltpu.sync_copy(stage_v, tab_sh.at[sl])
    pltpu.sync_copy(zeros_hbm.at[sl], stage_v)
    pltpu.sync_copy(stage_v, acc_sh.at[sl])
    plsc.subcore_barrier()

    base_row = wid * (EPT // CH)

    def blk(g, _):
        row = base_row + g * KI1
        pltpu.sync_copy(src_hbm.at[pl.ds(row, KI1)], sidx_v)
        pltpu.sync_copy(dst_hbm.at[pl.ds(row, KI1)], didx_v)
        gds = [
            pltpu.async_copy(tab_sh.at[sidx_v.at[j]], val_v.at[j], gsem)
            for j in range(KI1)
        ]
        for cp in gds:
            cp.wait()
        sds = [
            pltpu.async_copy(val_v.at[j], acc_sh.at[didx_v.at[j]], ssem,
                             add=True)
            for j in range(KI1)
        ]
        for cp in sds:
            cp.wait()
        return ()

    lax.fori_loop(0, NBLK1, blk, (), unroll=False)
    plsc.subcore_barrier()
    pltpu.sync_copy(acc_sh.at[sl], stage_v)
    pltpu.sync_copy(stage_v, accp_hbm.at[c, sl])


def _sc_edge2_body(src_hbm, dst_hbm, tab0_hbm, tab1_hbm, zeros_hbm, accp_hbm,
                   tab0_sh, tab1_sh, acc0_sh, acc1_sh,
                   sidx_v, didx_v, val0_v, val1_v, stage_v, gsem, ssem):
    c = lax.axis_index("c")
    s = lax.axis_index("s")
    wid = c * NS + s
    sl = pl.ds(s * SLICE, SLICE)
    pltpu.sync_copy(tab0_hbm.at[sl], stage_v)
    pltpu.sync_copy(stage_v, tab0_sh.at[sl])
    pltpu.sync_copy(tab1_hbm.at[sl], stage_v)
    pltpu.sync_copy(stage_v, tab1_sh.at[sl])
    pltpu.sync_copy(zeros_hbm.at[sl], stage_v)
    pltpu.sync_copy(stage_v, acc0_sh.at[sl])
    pltpu.sync_copy(stage_v, acc1_sh.at[sl])
    plsc.subcore_barrier()

    base_row = wid * (EPT // CH)

    def blk(g, _):
        row = base_row + g * KI3
        pltpu.sync_copy(src_hbm.at[pl.ds(row, KI3)], sidx_v)
        pltpu.sync_copy(dst_hbm.at[pl.ds(row, KI3)], didx_v)
        gds = []
        for j in range(KI3):
            gds.append(
                pltpu.async_copy(tab0_sh.at[sidx_v.at[j]], val0_v.at[j], gsem))
            gds.append(
                pltpu.async_copy(tab1_sh.at[sidx_v.at[j]], val1_v.at[j], gsem))
        for cp in gds:
            cp.wait()
        sds = []
        for j in range(KI3):
            sds.append(
                pltpu.async_copy(val0_v.at[j], acc0_sh.at[didx_v.at[j]], ssem,
                                 add=True))
            sds.append(
                pltpu.async_copy(val1_v.at[j], acc1_sh.at[didx_v.at[j]], ssem,
                                 add=True))
        for cp in sds:
            cp.wait()
        return ()

    lax.fori_loop(0, NBLK3, blk, (), unroll=False)
    plsc.subcore_barrier()
    pltpu.sync_copy(acc0_sh.at[sl], stage_v)
    pltpu.sync_copy(stage_v, accp_hbm.at[c, 0, sl])
    pltpu.sync_copy(acc1_sh.at[sl], stage_v)
    pltpu.sync_copy(stage_v, accp_hbm.at[c, 1, sl])


@functools.lru_cache(maxsize=None)
def _sc_kernels():
    mesh = plsc.VectorSubcoreMesh(core_axis_name="c", subcore_axis_name="s",
                                  num_cores=NC, num_subcores=NS)
    sc_deg = pl.kernel(
        _sc_deg_body,
        out_type=jax.ShapeDtypeStruct((NC, N_PAD), _f32),
        mesh=mesh,
        scratch_types=[
            pltpu.VMEM_SHARED((N_PAD,), _f32),
            pltpu.VMEM((KI1, CH), jnp.int32),
            pltpu.VMEM((CH,), _f32),
            pltpu.VMEM((SLICE,), _f32),
            pltpu.SemaphoreType.DMA,
        ],
    )
    sc_edge1 = pl.kernel(
        _sc_edge1_body,
        out_type=jax.ShapeDtypeStruct((NC, N_PAD), _f32),
        mesh=mesh,
        scratch_types=[
            pltpu.VMEM_SHARED((N_PAD,), _f32),
            pltpu.VMEM_SHARED((N_PAD,), _f32),
            pltpu.VMEM((KI1, CH), jnp.int32),
            pltpu.VMEM((KI1, CH), jnp.int32),
            pltpu.VMEM((KI1, CH), _f32),
            pltpu.VMEM((SLICE,), _f32),
            pltpu.SemaphoreType.DMA,
            pltpu.SemaphoreType.DMA,
        ],
    )
    sc_edge2 = pl.kernel(
        _sc_edge2_body,
        out_type=jax.ShapeDtypeStruct((NC, 2, N_PAD), _f32),
        mesh=mesh,
        scratch_types=[
            pltpu.VMEM_SHARED((N_PAD,), _f32),
            pltpu.VMEM_SHARED((N_PAD,), _f32),
            pltpu.VMEM_SHARED((N_PAD,), _f32),
            pltpu.VMEM_SHARED((N_PAD,), _f32),
            pltpu.VMEM((KI3, CH), jnp.int32),
            pltpu.VMEM((KI3, CH), jnp.int32),
            pltpu.VMEM((KI3, CH), _f32),
            pltpu.VMEM((KI3, CH), _f32),
            pltpu.VMEM((SLICE,), _f32),
            pltpu.SemaphoreType.DMA,
            pltpu.SemaphoreType.DMA,
        ],
    )
    return sc_deg, sc_edge1, sc_edge2


def _stage_a_body(degp_ref, x_ref, dinv_ref, xs_ref):
    d = degp_ref[0] + degp_ref[1] + 1.0
    dinv = lax.rsqrt(d)
    dinv_ref[...] = dinv
    xs_ref[...] = x_ref[...] * dinv


_stage_a = pl.pallas_call(
    _stage_a_body,
    out_shape=[jax.ShapeDtypeStruct((N_PAD // 128, 128), _f32)] * 2,
)


def _stage_b_body(a1p_ref, xs_ref, dinv_ref, w1_ref, b1_ref, w2_ref,
                  z0_ref, z1_ref):
    dinv = dinv_ref[...]
    y = dinv * (a1p_ref[0] + a1p_ref[1] + xs_ref[...])
    z0 = jnp.zeros_like(y)
    z1 = jnp.zeros_like(y)
    for k in range(16):
        h = jnp.maximum(y * w1_ref[0, k] + b1_ref[0, k], 0.0)
        z0 += h * w2_ref[k, 0]
        z1 += h * w2_ref[k, 1]
    z0_ref[...] = z0 * dinv
    z1_ref[...] = z1 * dinv


_stage_b = pl.pallas_call(
    _stage_b_body,
    in_specs=[
        pl.BlockSpec(),
        pl.BlockSpec(),
        pl.BlockSpec(),
        pl.BlockSpec(memory_space=pltpu.SMEM),
        pl.BlockSpec(memory_space=pltpu.SMEM),
        pl.BlockSpec(memory_space=pltpu.SMEM),
    ],
    out_shape=[jax.ShapeDtypeStruct((N_PAD // 128, 128), _f32)] * 2,
)


def _stage_c_body(a20_ref, a21_ref, zs0_ref, zs1_ref, dinv_ref, b2_ref,
                  o0_ref, o1_ref):
    dinv = dinv_ref[...]
    o0 = dinv * (a20_ref[0] + a20_ref[1] + zs0_ref[...]) + b2_ref[0, 0]
    o1 = dinv * (a21_ref[0] + a21_ref[1] + zs1_ref[...]) + b2_ref[0, 1]
    m = jnp.maximum(o0, o1)
    lse = m + jnp.log(jnp.exp(o0 - m) + jnp.exp(o1 - m))
    o0_ref[...] = o0 - lse
    o1_ref[...] = o1 - lse


_stage_c = pl.pallas_call(
    _stage_c_body,
    in_specs=[
        pl.BlockSpec(),
        pl.BlockSpec(),
        pl.BlockSpec(),
        pl.BlockSpec(),
        pl.BlockSpec(),
        pl.BlockSpec(memory_space=pltpu.SMEM),
    ],
    out_shape=[jax.ShapeDtypeStruct((N_PAD // 128, 128), _f32)] * 2,
)


def kernel(x, edge_index, W1, b1, W2, b2):
    sc_deg, sc_edge1, sc_edge2 = _sc_kernels()
    nrows = N_PAD // 128
    src = edge_index[0].astype(jnp.int32)
    dst = edge_index[1].astype(jnp.int32)
    pad = jnp.full((E_PAD - N_EDGES,), PAD_NODE, jnp.int32)
    src2d = jnp.concatenate([src, pad]).reshape(E_ROWS, CH)
    dst2d = jnp.concatenate([dst, pad]).reshape(E_ROWS, CH)

    xpad = jnp.zeros((N_PAD,), _f32).at[:N_NODES].set(x[:, 0])
    zeros1 = jnp.zeros((N_PAD,), _f32)
    ones = jnp.ones((CH,), _f32)

    degp = sc_deg(dst2d, ones, zeros1)
    dinv, xs = _stage_a(degp.reshape(NC, nrows, 128),
                        xpad.reshape(nrows, 128))

    acc1p = sc_edge1(src2d, dst2d, xs.reshape(N_PAD), zeros1)
    z0, z1 = _stage_b(acc1p.reshape(NC, nrows, 128), xs, dinv,
                      W1, b1.reshape(1, 16), W2)

    acc2p = sc_edge2(src2d, dst2d, z0.reshape(N_PAD), z1.reshape(N_PAD),
                     zeros1)

    a20 = acc2p[:, 0, :].reshape(NC, nrows, 128)
    a21 = acc2p[:, 1, :].reshape(NC, nrows, 128)
    o0, o1 = _stage_c(a20, a21, z0, z1, dinv, b2.reshape(1, 2))

    return jnp.stack(
        [o0.reshape(N_PAD)[:N_NODES], o1.reshape(N_PAD)[:N_NODES]], axis=1)
